# Initial kernel scaffold; baseline (speedup 1.0000x reference)
import jax
import jax.numpy as jnp
from jax.experimental import pallas as pl


def kernel(x, edge_index, W1, b1, W2, b2):
    raise NotImplementedError("write your pallas kernel here")


# speedup vs baseline: 48.8782x; 48.8782x over previous
import functools

import jax
import jax.numpy as jnp
from jax import lax
from jax.experimental import pallas as pl
from jax.experimental.pallas import tpu as pltpu
from jax.experimental.pallas import tpu_sc as plsc

N = 10000
E = 320000
E2 = E + N
DIN = 128
DH = 64
NC, NS, L = 2, 16, 16
NW = NC * NS
CH = 128
NCHW = 82
NCHT = NCHW * NW
EP = NCHT * CH
EWP = NCHW * CH
NPAD = 10240
TRASH = N
RPT = NPAD // NS
NSTG = N // NS
ZR = 64

_SC_PARAMS = pltpu.CompilerParams(needs_layout_passes=False,
                                  use_tc_tiling_on_sc=False)
_ONES_DN = (((0,), (0,)), ((), ()))


def _wid():
    return lax.axis_index("c") * NS + lax.axis_index("s")


def _dis_col(hist):
    deg = lax.dot_general(hist, jnp.ones((NW, 1), jnp.float32), _ONES_DN,
                          preferred_element_type=jnp.float32,
                          precision=lax.Precision.HIGHEST)
    return lax.rsqrt(jnp.maximum(deg, 1.0))


def _deg_body(dst_hbm, out_hbm, dst_v, hist):
    wid = _wid()
    pltpu.sync_copy(dst_hbm.at[wid], dst_v)

    def zero(i, _):
        for u in range(4):
            hist[pl.ds(pl.multiple_of((4 * i + u) * L, L), L)] = (
                jnp.zeros((L,), jnp.float32))
        return 0

    lax.fori_loop(0, NPAD // L // 4, zero, 0)
    ones = jnp.ones((L,), jnp.float32)

    def step(i, _):
        for u in range(4):
            d16 = dst_v[pl.ds(pl.multiple_of((4 * i + u) * L, L), L)]
            plsc.addupdate_scatter(hist, [d16], ones)
        return 0

    lax.fori_loop(0, EWP // L // 4, step, 0)
    pltpu.sync_copy(hist, out_hbm.at[wid])


@functools.cache
def _deg_call():
    return pl.kernel(
        _deg_body,
        out_type=jax.ShapeDtypeStruct((NW, NPAD), jnp.float32),
        mesh=plsc.VectorSubcoreMesh(core_axis_name="c", subcore_axis_name="s",
                                    num_cores=NC, num_subcores=NS),
        compiler_params=_SC_PARAMS,
        scratch_types=[
            pltpu.VMEM((EWP,), jnp.int32),
            pltpu.VMEM((NPAD,), jnp.float32),
        ],
    )


def _agg_body(g1_hbm, src_hbm, dst_hbm, out_hbm,
              src_v, dst_v, rows0, rows1, zb, g1_s, accum, sem0, sem1):
    cid = lax.axis_index("c")
    sid = lax.axis_index("s")

    pltpu.sync_copy(g1_hbm.at[pl.ds(sid * NSTG, NSTG)],
                    g1_s.at[pl.ds(sid * NSTG, NSTG)])

    def zzero(i, _):
        r = i // (DH // L)
        c = i % (DH // L)
        zb[r, pl.ds(pl.multiple_of(c * L, L), L)] = (
            jnp.zeros((L,), jnp.float32))
        return 0

    lax.fori_loop(0, ZR * (DH // L), zzero, 0)

    def zcopy(r, _):
        pltpu.sync_copy(zb, accum.at[pl.ds(sid * RPT + r * ZR, ZR)])
        return 0

    lax.fori_loop(0, RPT // ZR, zcopy, 0)
    plsc.subcore_barrier()

    def edge_run(base, nch):
        pltpu.sync_copy(src_hbm.at[pl.ds(base, nch)], src_v.at[pl.ds(0, nch)])
        pltpu.sync_copy(dst_hbm.at[pl.ds(base, nch)], dst_v.at[pl.ds(0, nch)])
        pltpu.async_copy(g1_s.at[src_v.at[0]], rows0, sem0)
        pltpu.async_copy(g1_s.at[src_v.at[1]], rows1, sem1)

        def step(k, _):
            j0 = 2 * k
            pltpu.make_async_copy(g1_s.at[src_v.at[j0]], rows0, sem0).wait()
            pltpu.sync_copy(rows0, accum.at[dst_v.at[j0]], add=True)
            pltpu.async_copy(g1_s.at[src_v.at[j0 + 2]], rows0, sem0)
            pltpu.make_async_copy(g1_s.at[src_v.at[j0 + 1]], rows1, sem1).wait()
            pltpu.sync_copy(rows1, accum.at[dst_v.at[j0 + 1]], add=True)
            pltpu.async_copy(g1_s.at[src_v.at[j0 + 3]], rows1, sem1)
            return 0

        lax.fori_loop(0, nch // 2 - 1, step, 0)
        jlast = nch - 2
        pltpu.make_async_copy(g1_s.at[src_v.at[jlast]], rows0, sem0).wait()
        pltpu.sync_copy(rows0, accum.at[dst_v.at[jlast]], add=True)
        pltpu.make_async_copy(g1_s.at[src_v.at[jlast + 1]], rows1, sem1).wait()
        pltpu.sync_copy(rows1, accum.at[dst_v.at[jlast + 1]], add=True)

    edge_run((cid * NS + sid) * NCHW, NCHW)

    plsc.subcore_barrier()
    pltpu.sync_copy(accum.at[pl.ds(sid * RPT, RPT)],
                    out_hbm.at[cid, pl.ds(sid * RPT, RPT)])


@functools.cache
def _agg_call():
    return pl.kernel(
        _agg_body,
        out_type=jax.ShapeDtypeStruct((NC, NPAD, DH), jnp.float32),
        mesh=plsc.VectorSubcoreMesh(core_axis_name="c", subcore_axis_name="s",
                                    num_cores=NC, num_subcores=NS),
        compiler_params=_SC_PARAMS,
        scratch_types=[
            pltpu.VMEM((NCHW, CH), jnp.int32),
            pltpu.VMEM((NCHW, CH), jnp.int32),
            pltpu.VMEM((CH, DH), jnp.float32),
            pltpu.VMEM((CH, DH), jnp.float32),
            pltpu.VMEM((ZR, DH), jnp.float32),
            pltpu.VMEM_SHARED((N, DH), jnp.float32),
            pltpu.VMEM_SHARED((NPAD, DH), jnp.float32),
            pltpu.SemaphoreType.DMA,
            pltpu.SemaphoreType.DMA,
        ],
    )


def _seg_body(g2_hbm, src_hbm, dst_hbm, out_hbm, g2_v, src_v, dst_v, acc):
    wid = _wid()
    pltpu.sync_copy(g2_hbm, g2_v)
    pltpu.sync_copy(src_hbm.at[wid], src_v)
    pltpu.sync_copy(dst_hbm.at[wid], dst_v)

    def zero(i, _):
        for u in range(4):
            acc[pl.ds(pl.multiple_of((4 * i + u) * L, L), L)] = (
                jnp.zeros((L,), jnp.float32))
        return 0

    lax.fori_loop(0, NPAD // L // 4, zero, 0)

    def step(i, _):
        for u in range(4):
            s16 = src_v[pl.ds(pl.multiple_of((4 * i + u) * L, L), L)]
            d16 = dst_v[pl.ds(pl.multiple_of((4 * i + u) * L, L), L)]
            row = lax.shift_right_logical(s16, 7)
            col = jnp.bitwise_and(s16, 127)
            vals = plsc.load_gather(g2_v, [row, col])
            plsc.addupdate_scatter(acc, [d16], vals)
        return 0

    lax.fori_loop(0, EWP // L // 4, step, 0)
    pltpu.sync_copy(acc, out_hbm.at[wid])


@functools.cache
def _seg_call():
    return pl.kernel(
        _seg_body,
        out_type=jax.ShapeDtypeStruct((NW, NPAD), jnp.float32),
        mesh=plsc.VectorSubcoreMesh(core_axis_name="c", subcore_axis_name="s",
                                    num_cores=NC, num_subcores=NS),
        compiler_params=_SC_PARAMS,
        scratch_types=[
            pltpu.VMEM((NPAD // CH, CH), jnp.float32),
            pltpu.VMEM((EWP,), jnp.int32),
            pltpu.VMEM((EWP,), jnp.int32),
            pltpu.VMEM((NPAD,), jnp.float32),
        ],
    )


def _lin1_body(x_ref, w1_ref, hist_ref, g1_ref, disr_ref):
    hist = hist_ref[...]
    deg_row = jnp.sum(hist, axis=0, keepdims=True)
    disr_ref[...] = lax.rsqrt(jnp.maximum(deg_row, 1.0))
    dis = _dis_col(hist)
    h1 = jnp.dot(x_ref[...], w1_ref[...],
                 preferred_element_type=jnp.float32,
                 precision=lax.Precision.HIGHEST)
    g1_ref[...] = dis[:N] * h1


def _lin1(x, w1, hist):
    return pl.pallas_call(
        _lin1_body,
        out_shape=(
            jax.ShapeDtypeStruct((N, DH), jnp.float32),
            jax.ShapeDtypeStruct((1, NPAD), jnp.float32),
        ),
    )(x, w1, hist)


def _lin2_body(u_ref, hist_ref, b1_ref, w2_ref, g2_ref):
    dis = _dis_col(hist_ref[...])
    u = u_ref[0, :N, :] + u_ref[1, :N, :]
    m1 = dis[:N] * u + b1_ref[...][None, :]
    a1 = jnp.maximum(m1, 0.0)
    h2 = jnp.dot(a1, w2_ref[...],
                 preferred_element_type=jnp.float32,
                 precision=lax.Precision.HIGHEST)
    g2_ref[:N] = dis[:N] * h2
    g2_ref[N:] = jnp.zeros((NPAD - N, 1), jnp.float32)


def _lin2(u_parts, hist, b1, w2):
    return pl.pallas_call(
        _lin2_body,
        out_shape=jax.ShapeDtypeStruct((NPAD, 1), jnp.float32),
    )(u_parts, hist, b1, w2)


def _fin_body(w_ref, disr_ref, b2_ref, out_ref):
    w_row = jnp.sum(w_ref[...], axis=0, keepdims=True)
    out_ref[...] = (disr_ref[:, :N] * w_row[:, :N]
                    + b2_ref[...][None, :])


def _fin(w_parts, dis_row, b2):
    return pl.pallas_call(
        _fin_body,
        out_shape=jax.ShapeDtypeStruct((1, N), jnp.float32),
    )(w_parts, dis_row, b2)


def kernel(x, edge_index, W1, b1, W2, b2):
    ei = edge_index.astype(jnp.int32)
    loop = jnp.arange(N, dtype=jnp.int32)
    pad = EP - E2
    src = jnp.concatenate([ei[0], loop, jnp.zeros((pad,), jnp.int32)])
    dst = jnp.concatenate([ei[1], loop, jnp.full((pad,), TRASH, jnp.int32)])
    src3 = src.reshape(NCHT, CH)
    dst3 = dst.reshape(NCHT, CH)
    src2 = src.reshape(NW, EWP)
    dst2 = dst.reshape(NW, EWP)

    hist = _deg_call()(dst2)
    g1, dis_row = _lin1(x, W1, hist)
    u_parts = _agg_call()(g1, src3, dst3)
    g2 = _lin2(u_parts, hist, b1, W2)
    w_parts = _seg_call()(g2.reshape(NPAD // CH, CH), src2, dst2)
    out = _fin(w_parts, dis_row, b2)
    return out[0]


# speedup vs baseline: 54.6812x; 1.1187x over previous
import functools

import jax
import jax.numpy as jnp
from jax import lax
from jax.experimental import pallas as pl
from jax.experimental.pallas import tpu as pltpu
from jax.experimental.pallas import tpu_sc as plsc

N = 10000
E = 320000
DIN = 128
DH = 64
NC, NS, L = 2, 16, 16
NW = NC * NS
CH = 128
NCHW = 80
NCHT = NCHW * NW
EP = NCHT * CH
EWP = NCHW * CH
NPAD = 10240
TRASH = N
RPT = NPAD // NS
NSTG = N // NS
ZR = 64

_SC_PARAMS = pltpu.CompilerParams(needs_layout_passes=False,
                                  use_tc_tiling_on_sc=False)
_ONES_DN = (((0,), (0,)), ((), ()))


def _wid():
    return lax.axis_index("c") * NS + lax.axis_index("s")


def _dis_col(hist):
    deg = lax.dot_general(hist, jnp.ones((NW, 1), jnp.float32), _ONES_DN,
                          preferred_element_type=jnp.float32)
    return lax.rsqrt(deg + 1.0)


def _deg_body(dst_hbm, out_hbm, dst_v, hist):
    wid = _wid()
    pltpu.sync_copy(dst_hbm.at[wid], dst_v)

    def zero(i, _):
        for u in range(4):
            hist[pl.ds(pl.multiple_of((4 * i + u) * L, L), L)] = (
                jnp.zeros((L,), jnp.float32))
        return 0

    lax.fori_loop(0, NPAD // L // 4, zero, 0)
    ones = jnp.ones((L,), jnp.float32)

    def step(i, _):
        for u in range(4):
            d16 = dst_v[pl.ds(pl.multiple_of((4 * i + u) * L, L), L)]
            plsc.addupdate_scatter(hist, [d16], ones)
        return 0

    lax.fori_loop(0, EWP // L // 4, step, 0)
    pltpu.sync_copy(hist, out_hbm.at[wid])


@functools.cache
def _deg_call():
    return pl.kernel(
        _deg_body,
        out_type=jax.ShapeDtypeStruct((NW, NPAD), jnp.float32),
        mesh=plsc.VectorSubcoreMesh(core_axis_name="c", subcore_axis_name="s",
                                    num_cores=NC, num_subcores=NS),
        compiler_params=_SC_PARAMS,
        scratch_types=[
            pltpu.VMEM((EWP,), jnp.int32),
            pltpu.VMEM((NPAD,), jnp.float32),
        ],
    )


def _agg_body(g1_hbm, src_hbm, dst_hbm, out_hbm,
              src_v, dst_v, rows0, rows1, zb, g1_s, accum, sem0, sem1):
    cid = lax.axis_index("c")
    sid = lax.axis_index("s")

    pltpu.sync_copy(g1_hbm.at[pl.ds(sid * NSTG, NSTG)],
                    g1_s.at[pl.ds(sid * NSTG, NSTG)])

    def zzero(i, _):
        r = i // (DH // L)
        c = i % (DH // L)
        zb[r, pl.ds(pl.multiple_of(c * L, L), L)] = (
            jnp.zeros((L,), jnp.float32))
        return 0

    lax.fori_loop(0, ZR * (DH // L), zzero, 0)

    def zcopy(r, _):
        pltpu.sync_copy(zb, accum.at[pl.ds(sid * RPT + r * ZR, ZR)])
        return 0

    lax.fori_loop(0, RPT // ZR, zcopy, 0)
    plsc.subcore_barrier()

    def edge_run(base, nch):
        pltpu.sync_copy(src_hbm.at[pl.ds(base, nch)], src_v.at[pl.ds(0, nch)])
        pltpu.sync_copy(dst_hbm.at[pl.ds(base, nch)], dst_v.at[pl.ds(0, nch)])
        pltpu.async_copy(g1_s.at[src_v.at[0]], rows0, sem0)
        pltpu.async_copy(g1_s.at[src_v.at[1]], rows1, sem1)

        def step(k, _):
            j0 = 2 * k
            pltpu.make_async_copy(g1_s.at[src_v.at[j0]], rows0, sem0).wait()
            pltpu.sync_copy(rows0, accum.at[dst_v.at[j0]], add=True)
            pltpu.async_copy(g1_s.at[src_v.at[j0 + 2]], rows0, sem0)
            pltpu.make_async_copy(g1_s.at[src_v.at[j0 + 1]], rows1, sem1).wait()
            pltpu.sync_copy(rows1, accum.at[dst_v.at[j0 + 1]], add=True)
            pltpu.async_copy(g1_s.at[src_v.at[j0 + 3]], rows1, sem1)
            return 0

        lax.fori_loop(0, nch // 2 - 1, step, 0)
        jlast = nch - 2
        pltpu.make_async_copy(g1_s.at[src_v.at[jlast]], rows0, sem0).wait()
        pltpu.sync_copy(rows0, accum.at[dst_v.at[jlast]], add=True)
        pltpu.make_async_copy(g1_s.at[src_v.at[jlast + 1]], rows1, sem1).wait()
        pltpu.sync_copy(rows1, accum.at[dst_v.at[jlast + 1]], add=True)

    edge_run((cid * NS + sid) * NCHW, NCHW)

    plsc.subcore_barrier()
    pltpu.sync_copy(accum.at[pl.ds(sid * RPT, RPT)],
                    out_hbm.at[cid, pl.ds(sid * RPT, RPT)])


@functools.cache
def _agg_call():
    return pl.kernel(
        _agg_body,
        out_type=jax.ShapeDtypeStruct((NC, NPAD, DH), jnp.float32),
        mesh=plsc.VectorSubcoreMesh(core_axis_name="c", subcore_axis_name="s",
                                    num_cores=NC, num_subcores=NS),
        compiler_params=_SC_PARAMS,
        scratch_types=[
            pltpu.VMEM((NCHW, CH), jnp.int32),
            pltpu.VMEM((NCHW, CH), jnp.int32),
            pltpu.VMEM((CH, DH), jnp.float32),
            pltpu.VMEM((CH, DH), jnp.float32),
            pltpu.VMEM((ZR, DH), jnp.float32),
            pltpu.VMEM_SHARED((N, DH), jnp.float32),
            pltpu.VMEM_SHARED((NPAD, DH), jnp.float32),
            pltpu.SemaphoreType.DMA,
            pltpu.SemaphoreType.DMA,
        ],
    )


def _seg_body(g2_hbm, src_hbm, dst_hbm, out_hbm, g2_v, src_v, dst_v, acc):
    wid = _wid()
    pltpu.sync_copy(g2_hbm, g2_v)
    pltpu.sync_copy(src_hbm.at[wid], src_v)
    pltpu.sync_copy(dst_hbm.at[wid], dst_v)

    def zero(i, _):
        for u in range(4):
            acc[pl.ds(pl.multiple_of((4 * i + u) * L, L), L)] = (
                jnp.zeros((L,), jnp.float32))
        return 0

    lax.fori_loop(0, NPAD // L // 4, zero, 0)

    def step(i, _):
        for u in range(4):
            s16 = src_v[pl.ds(pl.multiple_of((4 * i + u) * L, L), L)]
            d16 = dst_v[pl.ds(pl.multiple_of((4 * i + u) * L, L), L)]
            row = lax.shift_right_logical(s16, 7)
            col = jnp.bitwise_and(s16, 127)
            vals = plsc.load_gather(g2_v, [row, col])
            plsc.addupdate_scatter(acc, [d16], vals)
        return 0

    lax.fori_loop(0, EWP // L // 4, step, 0)
    pltpu.sync_copy(acc, out_hbm.at[wid])


@functools.cache
def _seg_call():
    return pl.kernel(
        _seg_body,
        out_type=jax.ShapeDtypeStruct((NW, NPAD), jnp.float32),
        mesh=plsc.VectorSubcoreMesh(core_axis_name="c", subcore_axis_name="s",
                                    num_cores=NC, num_subcores=NS),
        compiler_params=_SC_PARAMS,
        scratch_types=[
            pltpu.VMEM((NPAD // CH, CH), jnp.float32),
            pltpu.VMEM((EWP,), jnp.int32),
            pltpu.VMEM((EWP,), jnp.int32),
            pltpu.VMEM((NPAD,), jnp.float32),
        ],
    )


def _lin1_body(x_ref, w1_ref, hist_ref, g1_ref, disr_ref):
    hist = hist_ref[...]
    deg_row = jnp.sum(hist, axis=0, keepdims=True)
    disr_ref[...] = lax.rsqrt(deg_row + 1.0)
    dis = _dis_col(hist)
    h1 = jnp.dot(x_ref[...], w1_ref[...],
                 preferred_element_type=jnp.float32)
    g1_ref[...] = dis[:N] * h1


def _lin1(x, w1, hist):
    return pl.pallas_call(
        _lin1_body,
        out_shape=(
            jax.ShapeDtypeStruct((N, DH), jnp.float32),
            jax.ShapeDtypeStruct((1, NPAD), jnp.float32),
        ),
    )(x, w1, hist)


def _lin2_body(u_ref, hist_ref, g1_ref, b1_ref, w2_ref, g2_ref):
    dis = _dis_col(hist_ref[...])
    u = u_ref[0, :N, :] + u_ref[1, :N, :] + g1_ref[...]
    m1 = dis[:N] * u + b1_ref[...][None, :]
    a1 = jnp.maximum(m1, 0.0)
    h2 = jnp.dot(a1, w2_ref[...],
                 preferred_element_type=jnp.float32)
    g2_ref[:N] = dis[:N] * h2
    g2_ref[N:] = jnp.zeros((NPAD - N, 1), jnp.float32)


def _lin2(u_parts, hist, g1, b1, w2):
    return pl.pallas_call(
        _lin2_body,
        out_shape=jax.ShapeDtypeStruct((NPAD, 1), jnp.float32),
    )(u_parts, hist, g1, b1, w2)


def _fin_body(w_ref, g2r_ref, disr_ref, b2_ref, out_ref):
    w_row = jnp.sum(w_ref[...], axis=0, keepdims=True)
    out_ref[...] = (disr_ref[:, :N] * (w_row[:, :N] + g2r_ref[:, :N])
                    + b2_ref[...][None, :])


def _fin(w_parts, g2_row, dis_row, b2):
    return pl.pallas_call(
        _fin_body,
        out_shape=jax.ShapeDtypeStruct((1, N), jnp.float32),
    )(w_parts, g2_row, dis_row, b2)


def kernel(x, edge_index, W1, b1, W2, b2):
    ei = edge_index.astype(jnp.int32)
    pad = EP - E
    src = jnp.pad(ei[0], (0, pad))
    dst = jnp.pad(ei[1], (0, pad), constant_values=TRASH)
    src3 = src.reshape(NCHT, CH)
    dst3 = dst.reshape(NCHT, CH)
    src2 = src.reshape(NW, EWP)
    dst2 = dst.reshape(NW, EWP)

    hist = _deg_call()(dst2)
    g1, dis_row = _lin1(x, W1, hist)
    u_parts = _agg_call()(g1, src3, dst3)
    g2 = _lin2(u_parts, hist, g1, b1, W2)
    w_parts = _seg_call()(g2.reshape(NPAD // CH, CH), src2, dst2)
    out = _fin(w_parts, g2.reshape(1, NPAD), dis_row, b2)
    return out[0]


# speedup vs baseline: 67.5428x; 1.2352x over previous
import functools

import jax
import jax.numpy as jnp
from jax import lax
from jax.experimental import pallas as pl
from jax.experimental.pallas import tpu as pltpu
from jax.experimental.pallas import tpu_sc as plsc

N = 10000
E = 320000
DIN = 128
DH = 64
NC, NS, L = 2, 16, 16
NW = NC * NS
CH = 128
NCHW = 80
NCHT = NCHW * NW
EP = NCHT * CH
EWP = NCHW * CH
NPAD = 10240
TRASH = N
RPT = NPAD // NS
NSTG = N // NS
ZR = 64

_SC_PARAMS = pltpu.CompilerParams(needs_layout_passes=False,
                                  use_tc_tiling_on_sc=False)
_ONES_DN = (((0,), (0,)), ((), ()))


def _wid():
    return lax.axis_index("c") * NS + lax.axis_index("s")


def _dis_col(hist):
    deg = lax.dot_general(hist, jnp.ones((NW, 1), jnp.float32), _ONES_DN,
                          preferred_element_type=jnp.float32)
    return lax.rsqrt(deg + 1.0)


def _deg_body(dst_hbm, out_hbm, dst_v, hist):
    wid = _wid()
    pltpu.sync_copy(dst_hbm.at[wid], dst_v)

    def zero(i, _):
        for u in range(4):
            hist[pl.ds(pl.multiple_of((4 * i + u) * L, L), L)] = (
                jnp.zeros((L,), jnp.float32))
        return 0

    lax.fori_loop(0, NPAD // L // 4, zero, 0)
    ones = jnp.ones((L,), jnp.float32)

    def step(i, _):
        for u in range(4):
            d16 = dst_v[pl.ds(pl.multiple_of((4 * i + u) * L, L), L)]
            plsc.addupdate_scatter(hist, [d16], ones)
        return 0

    lax.fori_loop(0, EWP // L // 4, step, 0)
    pltpu.sync_copy(hist, out_hbm.at[wid])


@functools.cache
def _deg_call():
    return pl.kernel(
        _deg_body,
        out_type=jax.ShapeDtypeStruct((NW, NPAD), jnp.float32),
        mesh=plsc.VectorSubcoreMesh(core_axis_name="c", subcore_axis_name="s",
                                    num_cores=NC, num_subcores=NS),
        compiler_params=_SC_PARAMS,
        scratch_types=[
            pltpu.VMEM((EWP,), jnp.int32),
            pltpu.VMEM((NPAD,), jnp.float32),
        ],
    )


def _agg_body(g1_hbm, src_hbm, dst_hbm, out_hbm,
              src_v, dst_v, rows0, rows1, zb, g1_s, accum, sem0, sem1):
    cid = lax.axis_index("c")
    sid = lax.axis_index("s")

    pltpu.sync_copy(g1_hbm.at[pl.ds(sid * NSTG, NSTG)],
                    g1_s.at[pl.ds(sid * NSTG, NSTG)])

    def zzero(i, _):
        r = i // (DH // 32)
        c = i % (DH // 32)
        zb[r, pl.ds(pl.multiple_of(c * 32, 32), 32)] = (
            jnp.zeros((32,), jnp.bfloat16))
        return 0

    lax.fori_loop(0, ZR * (DH // 32), zzero, 0)

    def zcopy(r, _):
        pltpu.sync_copy(zb, accum.at[pl.ds(sid * RPT + r * ZR, ZR)])
        return 0

    lax.fori_loop(0, RPT // ZR, zcopy, 0)
    plsc.subcore_barrier()

    def edge_run(base, nch):
        pltpu.sync_copy(src_hbm.at[pl.ds(base, nch)], src_v.at[pl.ds(0, nch)])
        pltpu.sync_copy(dst_hbm.at[pl.ds(base, nch)], dst_v.at[pl.ds(0, nch)])
        pltpu.async_copy(g1_s.at[src_v.at[0]], rows0, sem0)
        pltpu.async_copy(g1_s.at[src_v.at[1]], rows1, sem1)

        def step(k, _):
            j0 = 2 * k
            pltpu.make_async_copy(g1_s.at[src_v.at[j0]], rows0, sem0).wait()
            pltpu.sync_copy(rows0, accum.at[dst_v.at[j0]], add=True)
            pltpu.async_copy(g1_s.at[src_v.at[j0 + 2]], rows0, sem0)
            pltpu.make_async_copy(g1_s.at[src_v.at[j0 + 1]], rows1, sem1).wait()
            pltpu.sync_copy(rows1, accum.at[dst_v.at[j0 + 1]], add=True)
            pltpu.async_copy(g1_s.at[src_v.at[j0 + 3]], rows1, sem1)
            return 0

        lax.fori_loop(0, nch // 2 - 1, step, 0)
        jlast = nch - 2
        pltpu.make_async_copy(g1_s.at[src_v.at[jlast]], rows0, sem0).wait()
        pltpu.sync_copy(rows0, accum.at[dst_v.at[jlast]], add=True)
        pltpu.make_async_copy(g1_s.at[src_v.at[jlast + 1]], rows1, sem1).wait()
        pltpu.sync_copy(rows1, accum.at[dst_v.at[jlast + 1]], add=True)

    edge_run((cid * NS + sid) * NCHW, NCHW)

    plsc.subcore_barrier()
    pltpu.sync_copy(accum.at[pl.ds(sid * RPT, RPT)],
                    out_hbm.at[cid, pl.ds(sid * RPT, RPT)])


@functools.cache
def _agg_call():
    return pl.kernel(
        _agg_body,
        out_type=jax.ShapeDtypeStruct((NC, NPAD, DH), jnp.bfloat16),
        mesh=plsc.VectorSubcoreMesh(core_axis_name="c", subcore_axis_name="s",
                                    num_cores=NC, num_subcores=NS),
        compiler_params=_SC_PARAMS,
        scratch_types=[
            pltpu.VMEM((NCHW, CH), jnp.int32),
            pltpu.VMEM((NCHW, CH), jnp.int32),
            pltpu.VMEM((CH, DH), jnp.bfloat16),
            pltpu.VMEM((CH, DH), jnp.bfloat16),
            pltpu.VMEM((ZR, DH), jnp.bfloat16),
            pltpu.VMEM_SHARED((N, DH), jnp.bfloat16),
            pltpu.VMEM_SHARED((NPAD, DH), jnp.bfloat16),
            pltpu.SemaphoreType.DMA,
            pltpu.SemaphoreType.DMA,
        ],
    )


def _seg_body(g2_hbm, src_hbm, dst_hbm, out_hbm, g2_v, src_v, dst_v, acc):
    wid = _wid()
    pltpu.sync_copy(g2_hbm, g2_v)
    pltpu.sync_copy(src_hbm.at[wid], src_v)
    pltpu.sync_copy(dst_hbm.at[wid], dst_v)

    def zero(i, _):
        for u in range(4):
            acc[pl.ds(pl.multiple_of((4 * i + u) * L, L), L)] = (
                jnp.zeros((L,), jnp.float32))
        return 0

    lax.fori_loop(0, NPAD // L // 4, zero, 0)

    def step(i, _):
        for u in range(4):
            s16 = src_v[pl.ds(pl.multiple_of((4 * i + u) * L, L), L)]
            d16 = dst_v[pl.ds(pl.multiple_of((4 * i + u) * L, L), L)]
            row = lax.shift_right_logical(s16, 7)
            col = jnp.bitwise_and(s16, 127)
            vals = plsc.load_gather(g2_v, [row, col])
            plsc.addupdate_scatter(acc, [d16], vals)
        return 0

    lax.fori_loop(0, EWP // L // 4, step, 0)
    pltpu.sync_copy(acc, out_hbm.at[wid])


@functools.cache
def _seg_call():
    return pl.kernel(
        _seg_body,
        out_type=jax.ShapeDtypeStruct((NW, NPAD), jnp.float32),
        mesh=plsc.VectorSubcoreMesh(core_axis_name="c", subcore_axis_name="s",
                                    num_cores=NC, num_subcores=NS),
        compiler_params=_SC_PARAMS,
        scratch_types=[
            pltpu.VMEM((NPAD // CH, CH), jnp.float32),
            pltpu.VMEM((EWP,), jnp.int32),
            pltpu.VMEM((EWP,), jnp.int32),
            pltpu.VMEM((NPAD,), jnp.float32),
        ],
    )


def _lin1_body(x_ref, w1_ref, hist_ref, g1_ref, disr_ref):
    hist = hist_ref[...]
    deg_row = jnp.sum(hist, axis=0, keepdims=True)
    disr_ref[...] = lax.rsqrt(deg_row + 1.0)
    dis = _dis_col(hist)
    h1 = jnp.dot(x_ref[...], w1_ref[...],
                 preferred_element_type=jnp.float32)
    g1_ref[...] = (dis[:N] * h1).astype(jnp.bfloat16)


def _lin1(x, w1, hist):
    return pl.pallas_call(
        _lin1_body,
        out_shape=(
            jax.ShapeDtypeStruct((N, DH), jnp.bfloat16),
            jax.ShapeDtypeStruct((1, NPAD), jnp.float32),
        ),
    )(x, w1, hist)


def _lin2_body(u_ref, hist_ref, g1_ref, b1_ref, w2_ref, g2_ref):
    dis = _dis_col(hist_ref[...])
    u = (u_ref[0, :N, :].astype(jnp.float32)
         + u_ref[1, :N, :].astype(jnp.float32)
         + g1_ref[...].astype(jnp.float32))
    m1 = dis[:N] * u + b1_ref[...][None, :]
    a1 = jnp.maximum(m1, 0.0)
    h2 = jnp.dot(a1, w2_ref[...],
                 preferred_element_type=jnp.float32)
    g2_ref[:N] = dis[:N] * h2
    g2_ref[N:] = jnp.zeros((NPAD - N, 1), jnp.float32)


def _lin2(u_parts, hist, g1, b1, w2):
    return pl.pallas_call(
        _lin2_body,
        out_shape=jax.ShapeDtypeStruct((NPAD, 1), jnp.float32),
    )(u_parts, hist, g1, b1, w2)


def _fin_body(w_ref, g2r_ref, disr_ref, b2_ref, out_ref):
    w_row = jnp.sum(w_ref[...], axis=0, keepdims=True)
    out_ref[...] = (disr_ref[:, :N] * (w_row[:, :N] + g2r_ref[:, :N])
                    + b2_ref[...][None, :])


def _fin(w_parts, g2_row, dis_row, b2):
    return pl.pallas_call(
        _fin_body,
        out_shape=jax.ShapeDtypeStruct((1, N), jnp.float32),
    )(w_parts, g2_row, dis_row, b2)


def kernel(x, edge_index, W1, b1, W2, b2):
    ei = edge_index.astype(jnp.int32)
    pad = EP - E
    src = jnp.pad(ei[0], (0, pad))
    dst = jnp.pad(ei[1], (0, pad), constant_values=TRASH)
    src3 = src.reshape(NCHT, CH)
    dst3 = dst.reshape(NCHT, CH)
    src2 = src.reshape(NW, EWP)
    dst2 = dst.reshape(NW, EWP)

    hist = _deg_call()(dst2)
    g1, dis_row = _lin1(x, W1, hist)
    u_parts = _agg_call()(g1, src3, dst3)
    g2 = _lin2(u_parts, hist, g1, b1, W2)
    w_parts = _seg_call()(g2.reshape(NPAD // CH, CH), src2, dst2)
    out = _fin(w_parts, g2.reshape(1, NPAD), dis_row, b2)
    return out[0]


# speedup vs baseline: 67.8694x; 1.0048x over previous
import functools

import jax
import jax.numpy as jnp
from jax import lax
from jax.experimental import pallas as pl
from jax.experimental.pallas import tpu as pltpu
from jax.experimental.pallas import tpu_sc as plsc

N = 10000
E = 320000
DIN = 128
DH = 64
NC, NS, L = 2, 16, 16
NW = NC * NS
CH = 128
NCHW = 80
NCHT = NCHW * NW
EP = NCHT * CH
EWP = NCHW * CH
NPAD = 10240
TRASH = N
RPT = NPAD // NS
RPTR = NPAD // CH // NS
NSTG = N // NS
ZR = 64

_SC_PARAMS = pltpu.CompilerParams(needs_layout_passes=False,
                                  use_tc_tiling_on_sc=False)
_ONES_DN = (((0,), (0,)), ((), ()))


def _dis_col(hist):
    deg = lax.dot_general(hist, jnp.ones((NC, 1), jnp.float32), _ONES_DN,
                          preferred_element_type=jnp.float32)
    return lax.rsqrt(deg + 1.0)


def _zero2d(buf):
    def zero(i, _):
        for u in range(4):
            k = 4 * i + u
            buf[k // (CH // L),
                pl.ds(pl.multiple_of((k % (CH // L)) * L, L), L)] = (
                jnp.zeros((L,), jnp.float32))
        return 0

    lax.fori_loop(0, NPAD // L // 4, zero, 0)


def _combine(part, shacc, out_hbm, cid, sid):
    iota = jnp.arange(L, dtype=jnp.int32)
    for j in range(NPAD // CH // L):
        pltpu.sync_copy(part.at[pl.ds(j * L, L)], shacc.at[iota + j * L],
                        add=True)
    plsc.subcore_barrier()
    pltpu.sync_copy(shacc.at[pl.ds(sid * RPTR, RPTR)],
                    out_hbm.at[cid, pl.ds(sid * RPTR, RPTR)])


def _deg_body(dst_hbm, out_hbm, dst_v, hist, shacc):
    cid = lax.axis_index("c")
    sid = lax.axis_index("s")
    wid = cid * NS + sid
    pltpu.sync_copy(dst_hbm.at[wid], dst_v)
    _zero2d(hist)
    pltpu.sync_copy(hist.at[pl.ds(sid * RPTR, RPTR)],
                    shacc.at[pl.ds(sid * RPTR, RPTR)])
    plsc.subcore_barrier()
    ones = jnp.ones((L,), jnp.float32)

    def step(i, _):
        for u in range(4):
            d16 = dst_v[pl.ds(pl.multiple_of((4 * i + u) * L, L), L)]
            row = lax.shift_right_logical(d16, 7)
            col = jnp.bitwise_and(d16, 127)
            plsc.addupdate_scatter(hist, [row, col], ones)
        return 0

    lax.fori_loop(0, EWP // L // 4, step, 0)
    _combine(hist, shacc, out_hbm, cid, sid)


@functools.cache
def _deg_call():
    return pl.kernel(
        _deg_body,
        out_type=jax.ShapeDtypeStruct((NC, NPAD // CH, CH), jnp.float32),
        mesh=plsc.VectorSubcoreMesh(core_axis_name="c", subcore_axis_name="s",
                                    num_cores=NC, num_subcores=NS),
        compiler_params=_SC_PARAMS,
        scratch_types=[
            pltpu.VMEM((EWP,), jnp.int32),
            pltpu.VMEM((NPAD // CH, CH), jnp.float32),
            pltpu.VMEM_SHARED((NPAD // CH, CH), jnp.float32),
        ],
    )


def _agg_body(g1_hbm, src_hbm, dst_hbm, out_hbm,
              src_v, dst_v, rows0, rows1, zb, g1_s, accum, sem0, sem1):
    cid = lax.axis_index("c")
    sid = lax.axis_index("s")

    pltpu.sync_copy(g1_hbm.at[pl.ds(sid * NSTG, NSTG)],
                    g1_s.at[pl.ds(sid * NSTG, NSTG)])

    def zzero(i, _):
        r = i // (DH // 32)
        c = i % (DH // 32)
        zb[r, pl.ds(pl.multiple_of(c * 32, 32), 32)] = (
            jnp.zeros((32,), jnp.bfloat16))
        return 0

    lax.fori_loop(0, ZR * (DH // 32), zzero, 0)

    def zcopy(r, _):
        pltpu.sync_copy(zb, accum.at[pl.ds(sid * RPT + r * ZR, ZR)])
        return 0

    lax.fori_loop(0, RPT // ZR, zcopy, 0)
    plsc.subcore_barrier()

    def edge_run(base, nch):
        pltpu.sync_copy(src_hbm.at[pl.ds(base, nch)], src_v.at[pl.ds(0, nch)])
        pltpu.sync_copy(dst_hbm.at[pl.ds(base, nch)], dst_v.at[pl.ds(0, nch)])
        pltpu.async_copy(g1_s.at[src_v.at[0]], rows0, sem0)
        pltpu.async_copy(g1_s.at[src_v.at[1]], rows1, sem1)

        def step(k, _):
            j0 = 2 * k
            pltpu.make_async_copy(g1_s.at[src_v.at[j0]], rows0, sem0).wait()
            pltpu.sync_copy(rows0, accum.at[dst_v.at[j0]], add=True)
            pltpu.async_copy(g1_s.at[src_v.at[j0 + 2]], rows0, sem0)
            pltpu.make_async_copy(g1_s.at[src_v.at[j0 + 1]], rows1, sem1).wait()
            pltpu.sync_copy(rows1, accum.at[dst_v.at[j0 + 1]], add=True)
            pltpu.async_copy(g1_s.at[src_v.at[j0 + 3]], rows1, sem1)
            return 0

        lax.fori_loop(0, nch // 2 - 1, step, 0)
        jlast = nch - 2
        pltpu.make_async_copy(g1_s.at[src_v.at[jlast]], rows0, sem0).wait()
        pltpu.sync_copy(rows0, accum.at[dst_v.at[jlast]], add=True)
        pltpu.make_async_copy(g1_s.at[src_v.at[jlast + 1]], rows1, sem1).wait()
        pltpu.sync_copy(rows1, accum.at[dst_v.at[jlast + 1]], add=True)

    edge_run((cid * NS + sid) * NCHW, NCHW)

    plsc.subcore_barrier()
    pltpu.sync_copy(accum.at[pl.ds(sid * RPT, RPT)],
                    out_hbm.at[cid, pl.ds(sid * RPT, RPT)])


@functools.cache
def _agg_call():
    return pl.kernel(
        _agg_body,
        out_type=jax.ShapeDtypeStruct((NC, NPAD, DH), jnp.bfloat16),
        mesh=plsc.VectorSubcoreMesh(core_axis_name="c", subcore_axis_name="s",
                                    num_cores=NC, num_subcores=NS),
        compiler_params=_SC_PARAMS,
        scratch_types=[
            pltpu.VMEM((NCHW, CH), jnp.int32),
            pltpu.VMEM((NCHW, CH), jnp.int32),
            pltpu.VMEM((CH, DH), jnp.bfloat16),
            pltpu.VMEM((CH, DH), jnp.bfloat16),
            pltpu.VMEM((ZR, DH), jnp.bfloat16),
            pltpu.VMEM_SHARED((N, DH), jnp.bfloat16),
            pltpu.VMEM_SHARED((NPAD, DH), jnp.bfloat16),
            pltpu.SemaphoreType.DMA,
            pltpu.SemaphoreType.DMA,
        ],
    )


def _seg_body(g2_hbm, src_hbm, dst_hbm, out_hbm, g2_v, src_v, dst_v, acc,
              shacc):
    cid = lax.axis_index("c")
    sid = lax.axis_index("s")
    wid = cid * NS + sid
    pltpu.sync_copy(g2_hbm, g2_v)
    pltpu.sync_copy(src_hbm.at[wid], src_v)
    pltpu.sync_copy(dst_hbm.at[wid], dst_v)
    _zero2d(acc)
    pltpu.sync_copy(acc.at[pl.ds(sid * RPTR, RPTR)],
                    shacc.at[pl.ds(sid * RPTR, RPTR)])
    plsc.subcore_barrier()

    def step(i, _):
        for u in range(4):
            s16 = src_v[pl.ds(pl.multiple_of((4 * i + u) * L, L), L)]
            d16 = dst_v[pl.ds(pl.multiple_of((4 * i + u) * L, L), L)]
            srow = lax.shift_right_logical(s16, 7)
            scol = jnp.bitwise_and(s16, 127)
            drow = lax.shift_right_logical(d16, 7)
            dcol = jnp.bitwise_and(d16, 127)
            vals = plsc.load_gather(g2_v, [srow, scol])
            plsc.addupdate_scatter(acc, [drow, dcol], vals)
        return 0

    lax.fori_loop(0, EWP // L // 4, step, 0)
    _combine(acc, shacc, out_hbm, cid, sid)


@functools.cache
def _seg_call():
    return pl.kernel(
        _seg_body,
        out_type=jax.ShapeDtypeStruct((NC, NPAD // CH, CH), jnp.float32),
        mesh=plsc.VectorSubcoreMesh(core_axis_name="c", subcore_axis_name="s",
                                    num_cores=NC, num_subcores=NS),
        compiler_params=_SC_PARAMS,
        scratch_types=[
            pltpu.VMEM((NPAD // CH, CH), jnp.float32),
            pltpu.VMEM((EWP,), jnp.int32),
            pltpu.VMEM((EWP,), jnp.int32),
            pltpu.VMEM((NPAD // CH, CH), jnp.float32),
            pltpu.VMEM_SHARED((NPAD // CH, CH), jnp.float32),
        ],
    )


def _lin1_body(x_ref, w1_ref, hist_ref, g1_ref, disr_ref):
    hist = hist_ref[...]
    deg_row = jnp.sum(hist, axis=0, keepdims=True)
    disr_ref[...] = lax.rsqrt(deg_row + 1.0)
    dis = _dis_col(hist)
    h1 = jnp.dot(x_ref[...], w1_ref[...],
                 preferred_element_type=jnp.float32)
    g1_ref[...] = (dis[:N] * h1).astype(jnp.bfloat16)


def _lin1(x, w1, hist):
    return pl.pallas_call(
        _lin1_body,
        out_shape=(
            jax.ShapeDtypeStruct((N, DH), jnp.bfloat16),
            jax.ShapeDtypeStruct((1, NPAD), jnp.float32),
        ),
    )(x, w1, hist)


def _lin2_body(u_ref, hist_ref, g1_ref, b1_ref, w2_ref, g2_ref):
    dis = _dis_col(hist_ref[...])
    u = (u_ref[0, :N, :].astype(jnp.float32)
         + u_ref[1, :N, :].astype(jnp.float32)
         + g1_ref[...].astype(jnp.float32))
    m1 = dis[:N] * u + b1_ref[...][None, :]
    a1 = jnp.maximum(m1, 0.0)
    h2 = jnp.dot(a1, w2_ref[...],
                 preferred_element_type=jnp.float32)
    g2_ref[:N] = dis[:N] * h2
    g2_ref[N:] = jnp.zeros((NPAD - N, 1), jnp.float32)


def _lin2(u_parts, hist, g1, b1, w2):
    return pl.pallas_call(
        _lin2_body,
        out_shape=jax.ShapeDtypeStruct((NPAD, 1), jnp.float32),
    )(u_parts, hist, g1, b1, w2)


def _fin_body(w_ref, g2r_ref, disr_ref, b2_ref, out_ref):
    w_row = jnp.sum(w_ref[...], axis=0, keepdims=True)
    out_ref[...] = (disr_ref[:, :N] * (w_row[:, :N] + g2r_ref[:, :N])
                    + b2_ref[...][None, :])


def _fin(w_parts, g2_row, dis_row, b2):
    return pl.pallas_call(
        _fin_body,
        out_shape=jax.ShapeDtypeStruct((1, N), jnp.float32),
    )(w_parts, g2_row, dis_row, b2)


def kernel(x, edge_index, W1, b1, W2, b2):
    ei = edge_index.astype(jnp.int32)
    pad = EP - E
    src = jnp.pad(ei[0], (0, pad))
    dst = jnp.pad(ei[1], (0, pad), constant_values=TRASH)
    src3 = src.reshape(NCHT, CH)
    dst3 = dst.reshape(NCHT, CH)
    src2 = src.reshape(NW, EWP)
    dst2 = dst.reshape(NW, EWP)

    hist = _deg_call()(dst2).reshape(NC, NPAD)
    g1, dis_row = _lin1(x, W1, hist)
    u_parts = _agg_call()(g1, src3, dst3)
    g2 = _lin2(u_parts, hist, g1, b1, W2)
    w_parts = _seg_call()(g2.reshape(NPAD // CH, CH), src2, dst2)
    out = _fin(w_parts.reshape(NC, NPAD), g2.reshape(1, NPAD), dis_row, b2)
    return out[0]
